# baseline (speedup 1.0000x reference)
import functools

import jax
import jax.numpy as jnp
from jax import lax
from jax.experimental import pallas as pl
from jax.experimental.pallas import tpu as pltpu
from jax.experimental.pallas import tpu_sc as plsc

B = 16384
D = 64

_info = plsc.get_sparse_core_info()
_NC, _NS = _info.num_cores, _info.num_subcores
_NW = _NC * _NS
_BPW = B // _NW
_CHUNK = 128
_NCHUNK = _BPW // _CHUNK
_NBLK = _BPW // 16


def _dot_kernel(user_idx, content_idx, user_table, content_table,
                out_hbm, uidx_v, cidx_v, urows_v, crows_v, out_v, sem):
    wid = lax.axis_index("s") * _NC + lax.axis_index("c")

    pltpu.sync_copy(user_idx.at[pl.ds(wid * _NCHUNK, _NCHUNK)], uidx_v)
    pltpu.sync_copy(content_idx.at[pl.ds(wid * _NCHUNK, _NCHUNK)], cidx_v)

    copies = []
    for j in range(_NCHUNK):
        copies.append(pltpu.async_copy(
            user_table.at[uidx_v.at[j]],
            urows_v.at[pl.ds(j * _CHUNK, _CHUNK)], sem))
        copies.append(pltpu.async_copy(
            content_table.at[cidx_v.at[j]],
            crows_v.at[pl.ds(j * _CHUNK, _CHUNK)], sem))
    for c in copies:
        c.wait()

    lanes = lax.iota(jnp.int32, 16)

    def block(b, carry):
        rows = lanes + b * 16
        acc0 = jnp.zeros((16,), jnp.float32)
        acc1 = jnp.zeros((16,), jnp.float32)
        for d in range(0, D, 2):
            d0 = jnp.full((16,), d, jnp.int32)
            d1 = jnp.full((16,), d + 1, jnp.int32)
            acc0 += (plsc.load_gather(urows_v, [rows, d0])
                     * plsc.load_gather(crows_v, [rows, d0]))
            acc1 += (plsc.load_gather(urows_v, [rows, d1])
                     * plsc.load_gather(crows_v, [rows, d1]))
        out_v[pl.ds(b * 16, 16)] = acc0 + acc1
        return carry

    lax.fori_loop(0, _NBLK, block, 0)

    pltpu.sync_copy(out_v, out_hbm.at[pl.ds(wid * _BPW, _BPW)])


@jax.jit
def _run(user_idx2d, content_idx2d, user_table, content_table):
    mesh = plsc.VectorSubcoreMesh(core_axis_name="c", subcore_axis_name="s")
    f = functools.partial(
        pl.kernel, mesh=mesh,
        out_type=jax.ShapeDtypeStruct((B,), jnp.float32),
        compiler_params=pltpu.CompilerParams(
            needs_layout_passes=False, use_tc_tiling_on_sc=False),
        scratch_types=[
            pltpu.VMEM((_NCHUNK, _CHUNK), jnp.int32),
            pltpu.VMEM((_NCHUNK, _CHUNK), jnp.int32),
            pltpu.VMEM((_BPW, D), jnp.float32),
            pltpu.VMEM((_BPW, D), jnp.float32),
            pltpu.VMEM((_BPW,), jnp.float32),
            pltpu.SemaphoreType.DMA,
        ],
    )(_dot_kernel)
    return f(user_idx2d, content_idx2d, user_table, content_table)


def kernel(user, content, user_table, content_table):
    out = _run(user.reshape(B // _CHUNK, _CHUNK),
               content.reshape(B // _CHUNK, _CHUNK),
               user_table, content_table)
    return out.reshape(B, 1)
